# baseline (speedup 1.0000x reference)
import functools

import jax
import jax.numpy as jnp
from jax.experimental import pallas as pl
from jax.experimental.pallas import tpu as pltpu


def _round_up(x: int, m: int) -> int:
    return ((x + m - 1) // m) * m


_NBUF = 8
_OBUF = 2
_CORES = 2


def _gcn_manual_kernel(text_hbm, adj_hbm, w_ref, b_ref, out_hbm,
                       text_buf, h_buf, adj_buf, out_buf,
                       sem_text, sem_adj, sem_out, *, g, n, f_in):
    base = pl.program_id(0) * g

    def adj_copy(b, slot):
        return pltpu.make_async_copy(
            adj_hbm.at[base + b], adj_buf.at[slot], sem_adj.at[slot])

    def out_copy(b, slot):
        return pltpu.make_async_copy(
            out_buf.at[slot], out_hbm.at[base + b], sem_out.at[slot])

    text_cp = pltpu.make_async_copy(
        text_hbm.at[pl.ds(base, g)], text_buf, sem_text)
    text_cp.start()
    for b in range(min(_NBUF, g)):
        adj_copy(b, b % _NBUF).start()

    text_cp.wait()
    x = text_buf[...].reshape(g * n, f_in)
    h = jax.lax.dot_general(x, w_ref[...], (((1,), (1,)), ((), ())),
                            preferred_element_type=jnp.float32)
    h_buf[...] = h + b_ref[...]

    for b in range(g):
        slot = b % _NBUF
        adj_copy(b, slot).wait()
        adj_m = adj_buf[slot]
        agg = jnp.dot(adj_m, h_buf[pl.ds(b * n, n), :],
                      preferred_element_type=jnp.float32)
        denom = jnp.sum(adj_m, axis=1, keepdims=True) + 1.0
        inv = pl.reciprocal(denom, approx=False)
        oslot = b % _OBUF
        if b >= _OBUF:
            out_copy(b - _OBUF, oslot).wait()
        out_buf[oslot] = (agg * inv).astype(out_buf.dtype)
        out_copy(b, oslot).start()
        if b + _NBUF < g:
            adj_copy(b + _NBUF, slot).start()

    for k in range(min(_OBUF, g)):
        b = g - min(_OBUF, g) + k
        out_copy(b, b % _OBUF).wait()


def kernel(text, adj, weight, bias):
    B, N, F_in = text.shape
    F_out = weight.shape[0]

    N_pad = _round_up(N, 128)
    F_in_pad = _round_up(F_in, 128)
    F_out_pad = _round_up(F_out, 128)
    cores = _CORES if (B % _CORES == 0 and B <= 64) else 1
    B_pad = _round_up(B, cores)
    g = B_pad // cores

    f32 = jnp.float32
    text_p = jnp.pad(text.astype(f32),
                     ((0, B_pad - B), (0, N_pad - N), (0, F_in_pad - F_in)))
    adj_p = jnp.pad(adj.astype(f32),
                    ((0, B_pad - B), (0, N_pad - N), (0, N_pad - N)))
    w_p = jnp.pad(weight.astype(f32),
                  ((0, F_out_pad - F_out), (0, F_in_pad - F_in)))
    b_p = jnp.pad(bias.astype(f32), (0, F_out_pad - F_out)).reshape(1, -1)

    body = functools.partial(_gcn_manual_kernel, g=g, n=N_pad, f_in=F_in_pad)
    out_p = pl.pallas_call(
        body,
        out_shape=jax.ShapeDtypeStruct((B_pad, N_pad, F_out_pad), text.dtype),
        grid=(cores,),
        in_specs=[
            pl.BlockSpec(memory_space=pl.ANY),
            pl.BlockSpec(memory_space=pl.ANY),
            pl.BlockSpec(memory_space=pltpu.MemorySpace.VMEM),
            pl.BlockSpec(memory_space=pltpu.MemorySpace.VMEM),
        ],
        out_specs=pl.BlockSpec(memory_space=pl.ANY),
        scratch_shapes=[
            pltpu.VMEM((g, N_pad, F_in_pad), f32),
            pltpu.VMEM((g * N_pad, F_out_pad), f32),
            pltpu.VMEM((_NBUF, N_pad, N_pad), f32),
            pltpu.VMEM((_OBUF, N_pad, F_out_pad), f32),
            pltpu.SemaphoreType.DMA,
            pltpu.SemaphoreType.DMA((_NBUF,)),
            pltpu.SemaphoreType.DMA((_OBUF,)),
        ],
        compiler_params=pltpu.CompilerParams(
            dimension_semantics=("parallel",)),
    )(text_p, adj_p, w_p, b_p)

    return out_p[:B, :N, :F_out]


# speedup vs baseline: 1.1278x; 1.1278x over previous
import functools

import jax
import jax.numpy as jnp
from jax.experimental import pallas as pl
from jax.experimental.pallas import tpu as pltpu


def _round_up(x: int, m: int) -> int:
    return ((x + m - 1) // m) * m


_SLAB = 4
_NBUF = 3
_OBUF = 2
_CORES = 2


def _gcn_manual_kernel(text_hbm, adj_hbm, w_ref, b_ref, out_hbm,
                       text_buf, h_buf, adj_buf, out_buf,
                       sem_text, sem_adj, sem_out, *, g, n, f_in):
    base = pl.program_id(0) * g
    n_slabs = g // _SLAB

    def adj_copy(s, slot):
        return pltpu.make_async_copy(
            adj_hbm.at[pl.ds(base + s * _SLAB, _SLAB)], adj_buf.at[slot],
            sem_adj.at[slot])

    def out_copy(s, slot):
        return pltpu.make_async_copy(
            out_buf.at[slot], out_hbm.at[pl.ds(base + s * _SLAB, _SLAB)],
            sem_out.at[slot])

    text_cp = pltpu.make_async_copy(
        text_hbm.at[pl.ds(base, g)], text_buf, sem_text)
    text_cp.start()
    for s in range(min(_NBUF, n_slabs)):
        adj_copy(s, s % _NBUF).start()

    text_cp.wait()
    x = text_buf[...].reshape(g * n, f_in)
    h = jax.lax.dot_general(x, w_ref[...], (((1,), (1,)), ((), ())),
                            preferred_element_type=jnp.float32)
    h_buf[...] = h + b_ref[...]

    for s in range(n_slabs):
        slot = s % _NBUF
        adj_copy(s, slot).wait()
        oslot = s % _OBUF
        if s >= _OBUF:
            out_copy(s - _OBUF, oslot).wait()
        for i in range(_SLAB):
            adj_m = adj_buf[slot, i]
            agg = jnp.dot(adj_m, h_buf[pl.ds((s * _SLAB + i) * n, n), :],
                          preferred_element_type=jnp.float32)
            denom = jnp.sum(adj_m, axis=1, keepdims=True) + 1.0
            inv = pl.reciprocal(denom, approx=False)
            out_buf[oslot, i] = (agg * inv).astype(out_buf.dtype)
        out_copy(s, oslot).start()
        if s + _NBUF < n_slabs:
            adj_copy(s + _NBUF, slot).start()

    for k in range(min(_OBUF, n_slabs)):
        s = n_slabs - min(_OBUF, n_slabs) + k
        out_copy(s, s % _OBUF).wait()


def kernel(text, adj, weight, bias):
    B, N, F_in = text.shape
    F_out = weight.shape[0]

    N_pad = _round_up(N, 128)
    F_in_pad = _round_up(F_in, 128)
    F_out_pad = _round_up(F_out, 128)
    group = _CORES * _SLAB
    cores = _CORES if (B % group == 0 and B <= 64) else 1
    B_pad = _round_up(B, cores * _SLAB)
    g = B_pad // cores

    f32 = jnp.float32
    text_p = jnp.pad(text.astype(f32),
                     ((0, B_pad - B), (0, N_pad - N), (0, F_in_pad - F_in)))
    adj_p = jnp.pad(adj.astype(f32),
                    ((0, B_pad - B), (0, N_pad - N), (0, N_pad - N)))
    w_p = jnp.pad(weight.astype(f32),
                  ((0, F_out_pad - F_out), (0, F_in_pad - F_in)))
    b_p = jnp.pad(bias.astype(f32), (0, F_out_pad - F_out)).reshape(1, -1)

    body = functools.partial(_gcn_manual_kernel, g=g, n=N_pad, f_in=F_in_pad)
    out_p = pl.pallas_call(
        body,
        out_shape=jax.ShapeDtypeStruct((B_pad, N_pad, F_out_pad), text.dtype),
        grid=(cores,),
        in_specs=[
            pl.BlockSpec(memory_space=pl.ANY),
            pl.BlockSpec(memory_space=pl.ANY),
            pl.BlockSpec(memory_space=pltpu.MemorySpace.VMEM),
            pl.BlockSpec(memory_space=pltpu.MemorySpace.VMEM),
        ],
        out_specs=pl.BlockSpec(memory_space=pl.ANY),
        scratch_shapes=[
            pltpu.VMEM((g, N_pad, F_in_pad), f32),
            pltpu.VMEM((g * N_pad, F_out_pad), f32),
            pltpu.VMEM((_NBUF, _SLAB, N_pad, N_pad), f32),
            pltpu.VMEM((_OBUF, _SLAB, N_pad, F_out_pad), f32),
            pltpu.SemaphoreType.DMA,
            pltpu.SemaphoreType.DMA((_NBUF,)),
            pltpu.SemaphoreType.DMA((_OBUF,)),
        ],
        compiler_params=pltpu.CompilerParams(
            dimension_semantics=("parallel",)),
    )(text_p, adj_p, w_p, b_p)

    return out_p[:B, :N, :F_out]


# speedup vs baseline: 1.2754x; 1.1309x over previous
import functools

import jax
import jax.numpy as jnp
from jax.experimental import pallas as pl
from jax.experimental.pallas import tpu as pltpu


def _round_up(x: int, m: int) -> int:
    return ((x + m - 1) // m) * m


_SLAB = 8
_NBUF = 2
_OBUF = 2
_CORES = 2


def _gcn_manual_kernel(text_hbm, adj_hbm, w_ref, b_ref, out_hbm,
                       text_buf, h_buf, adj_buf, out_buf,
                       sem_text, sem_adj, sem_out, *, g, n, f_in):
    base = pl.program_id(0) * g
    n_slabs = g // _SLAB

    def adj_copy(s, slot):
        return pltpu.make_async_copy(
            adj_hbm.at[pl.ds(base + s * _SLAB, _SLAB)], adj_buf.at[slot],
            sem_adj.at[slot])

    def out_copy(s, slot):
        return pltpu.make_async_copy(
            out_buf.at[slot], out_hbm.at[pl.ds(base + s * _SLAB, _SLAB)],
            sem_out.at[slot])

    text_cp = pltpu.make_async_copy(
        text_hbm.at[pl.ds(base, g)], text_buf, sem_text)
    text_cp.start()
    for s in range(min(_NBUF, n_slabs)):
        adj_copy(s, s % _NBUF).start()

    text_cp.wait()
    x = text_buf[...].reshape(g * n, f_in)
    h = jax.lax.dot_general(x, w_ref[...], (((1,), (1,)), ((), ())),
                            preferred_element_type=jnp.float32)
    h_buf[...] = h + b_ref[...]

    for s in range(n_slabs):
        slot = s % _NBUF
        adj_copy(s, slot).wait()
        oslot = s % _OBUF
        if s >= _OBUF:
            out_copy(s - _OBUF, oslot).wait()
        for i in range(_SLAB):
            adj_m = adj_buf[slot, i]
            agg = jnp.dot(adj_m, h_buf[pl.ds((s * _SLAB + i) * n, n), :],
                          preferred_element_type=jnp.float32)
            denom = jnp.sum(adj_m, axis=1, keepdims=True) + 1.0
            inv = pl.reciprocal(denom, approx=False)
            out_buf[oslot, i] = (agg * inv).astype(out_buf.dtype)
        out_copy(s, oslot).start()
        if s + _NBUF < n_slabs:
            adj_copy(s + _NBUF, slot).start()

    for k in range(min(_OBUF, n_slabs)):
        s = n_slabs - min(_OBUF, n_slabs) + k
        out_copy(s, s % _OBUF).wait()


def kernel(text, adj, weight, bias):
    B, N, F_in = text.shape
    F_out = weight.shape[0]

    N_pad = _round_up(N, 128)
    F_in_pad = _round_up(F_in, 128)
    F_out_pad = _round_up(F_out, 128)
    group = _CORES * _SLAB
    cores = _CORES if (B % group == 0 and B <= 64) else 1
    B_pad = _round_up(B, cores * _SLAB)
    g = B_pad // cores

    f32 = jnp.float32
    text_p = jnp.pad(text.astype(f32),
                     ((0, B_pad - B), (0, N_pad - N), (0, F_in_pad - F_in)))
    adj_p = jnp.pad(adj.astype(f32),
                    ((0, B_pad - B), (0, N_pad - N), (0, N_pad - N)))
    w_p = jnp.pad(weight.astype(f32),
                  ((0, F_out_pad - F_out), (0, F_in_pad - F_in)))
    b_p = jnp.pad(bias.astype(f32), (0, F_out_pad - F_out)).reshape(1, -1)

    body = functools.partial(_gcn_manual_kernel, g=g, n=N_pad, f_in=F_in_pad)
    out_p = pl.pallas_call(
        body,
        out_shape=jax.ShapeDtypeStruct((B_pad, N_pad, F_out_pad), text.dtype),
        grid=(cores,),
        in_specs=[
            pl.BlockSpec(memory_space=pl.ANY),
            pl.BlockSpec(memory_space=pl.ANY),
            pl.BlockSpec(memory_space=pltpu.MemorySpace.VMEM),
            pl.BlockSpec(memory_space=pltpu.MemorySpace.VMEM),
        ],
        out_specs=pl.BlockSpec(memory_space=pl.ANY),
        scratch_shapes=[
            pltpu.VMEM((g, N_pad, F_in_pad), f32),
            pltpu.VMEM((g * N_pad, F_out_pad), f32),
            pltpu.VMEM((_NBUF, _SLAB, N_pad, N_pad), f32),
            pltpu.VMEM((_OBUF, _SLAB, N_pad, F_out_pad), f32),
            pltpu.SemaphoreType.DMA,
            pltpu.SemaphoreType.DMA((_NBUF,)),
            pltpu.SemaphoreType.DMA((_OBUF,)),
        ],
        compiler_params=pltpu.CompilerParams(
            dimension_semantics=("parallel",)),
    )(text_p, adj_p, w_p, b_p)

    return out_p[:B, :N, :F_out]


# speedup vs baseline: 1.5392x; 1.2069x over previous
import functools

import jax
import jax.numpy as jnp
from jax.experimental import pallas as pl
from jax.experimental.pallas import tpu as pltpu


def _round_up(x: int, m: int) -> int:
    return ((x + m - 1) // m) * m


_BB = 8
_STEPS = 2


def _fused_gcn_kernel(text_ref, adj_ref, w_ref, b_ref, out_ref, h_ref,
                      *, bb, steps, n):
    f_in = w_ref.shape[1]
    j = pl.program_id(1)

    @pl.when(j == 0)
    def _compute_hidden():
        x = text_ref[...].reshape(bb * steps * n, f_in)
        h = jax.lax.dot_general(x, w_ref[...], (((1,), (1,)), ((), ())),
                                preferred_element_type=jnp.float32)
        h_ref[...] = h + b_ref[...]

    for i in range(bb):
        adj = adj_ref[i]
        h_i = h_ref[pl.ds((j * bb + i) * n, n), :]
        agg = jnp.dot(adj, h_i, preferred_element_type=jnp.float32)
        denom = jnp.sum(adj, axis=1, keepdims=True) + 1.0
        inv = pl.reciprocal(denom, approx=False)
        out_ref[i] = (agg * inv).astype(out_ref.dtype)


def kernel(text, adj, weight, bias):
    B, N, F_in = text.shape
    F_out = weight.shape[0]

    N_pad = _round_up(N, 128)
    F_in_pad = _round_up(F_in, 128)
    F_out_pad = _round_up(F_out, 128)
    group = _BB * _STEPS
    if B % group == 0:
        bb, steps = _BB, _STEPS
    else:
        bb, steps = 1, 1
    B_pad = _round_up(B, bb * steps)

    f32 = jnp.float32
    text_p = jnp.pad(text.astype(f32),
                     ((0, B_pad - B), (0, N_pad - N), (0, F_in_pad - F_in)))
    adj_p = jnp.pad(adj.astype(f32),
                    ((0, B_pad - B), (0, N_pad - N), (0, N_pad - N)))
    w_p = jnp.pad(weight.astype(f32),
                  ((0, F_out_pad - F_out), (0, F_in_pad - F_in)))
    b_p = jnp.pad(bias.astype(f32), (0, F_out_pad - F_out)).reshape(1, -1)

    body = functools.partial(_fused_gcn_kernel, bb=bb, steps=steps, n=N_pad)
    out_p = pl.pallas_call(
        body,
        out_shape=jax.ShapeDtypeStruct((B_pad, N_pad, F_out_pad), text.dtype),
        grid=(B_pad // (bb * steps), steps),
        in_specs=[
            pl.BlockSpec((bb * steps, N_pad, F_in_pad),
                         lambda i, j: (i, 0, 0)),
            pl.BlockSpec((bb, N_pad, N_pad),
                         lambda i, j, s=steps: (i * s + j, 0, 0)),
            pl.BlockSpec((F_out_pad, F_in_pad), lambda i, j: (0, 0)),
            pl.BlockSpec((1, F_out_pad), lambda i, j: (0, 0)),
        ],
        out_specs=pl.BlockSpec((bb, N_pad, F_out_pad),
                               lambda i, j, s=steps: (i * s + j, 0, 0)),
        scratch_shapes=[pltpu.VMEM((bb * steps * N_pad, F_out_pad), f32)],
        compiler_params=pltpu.CompilerParams(
            dimension_semantics=("parallel", "arbitrary")),
    )(text_p, adj_p, w_p, b_p)

    return out_p[:B, :N, :F_out]
